# baseline (speedup 1.0000x reference)
import jax
import jax.numpy as jnp
from jax.experimental import pallas as pl
from jax.experimental.pallas import tpu as pltpu

N = 128
D = 3 * 224 * 224
ROWS = D // 128
SPB = 8
STEPS = N // SPB
SPLITS = 3
RSPLIT = ROWS // SPLITS
REM = int(N * (1.0 - 0.1))


def _acc_kernel(*refs):
    xr1_refs = refs[0:SPLITS]
    xr2_refs = refs[SPLITS:2 * SPLITS]
    x_refs = refs[2 * SPLITS:3 * SPLITS]
    acc1_ref, acc2_ref = refs[3 * SPLITS], refs[3 * SPLITS + 1]
    p1 = jnp.zeros((SPB, 1, 1), jnp.float32)
    p2 = jnp.zeros((SPB, 1, 1), jnp.float32)
    for q in range(SPLITS):
        x = x_refs[q][...]
        d1 = xr1_refs[q][...] - x
        d2 = xr2_refs[q][...] - x
        p1 = p1 + jnp.sum(d1 * d1, axis=(1, 2), keepdims=True)
        p2 = p2 + jnp.sum(d2 * d2, axis=(1, 2), keepdims=True)
    acc1_ref[...] = p1
    acc2_ref[...] = p2


def _select_kernel(a1c_ref, a2c_ref, a1r_ref, a2r_ref, l1_ref, l2_ref):
    a1c = a1c_ref[...]
    a2c = a2c_ref[...]
    a1r = a1r_ref[...]
    a2r = a2r_ref[...]
    jidx = jax.lax.broadcasted_iota(jnp.int32, (N, N), 1)
    iidx = jax.lax.broadcasted_iota(jnp.int32, (N, N), 0)
    tie = jidx < iidx
    cmp2 = (a2r < a2c) | ((a2r == a2c) & tie)
    cmp1 = (a1r < a1c) | ((a1r == a1c) & tie)
    rank2 = jnp.sum(cmp2.astype(jnp.int32), axis=1, keepdims=True)
    rank1 = jnp.sum(cmp1.astype(jnp.int32), axis=1, keepdims=True)
    sel2 = rank2 < REM
    sel1 = rank1 < REM
    scale = 1.0 / (REM * D)
    l1_ref[...] = jnp.sum(jnp.where(sel2, a1c, 0.0), axis=0, keepdims=True) * scale
    l2_ref[...] = jnp.sum(jnp.where(sel1, a2c, 0.0), axis=0, keepdims=True) * scale


def kernel(xr1, xr2, x):
    xr1 = xr1.reshape(N, ROWS, 128)
    xr2 = xr2.reshape(N, ROWS, 128)
    x = x.reshape(N, ROWS, 128)

    def split_spec(q):
        return pl.BlockSpec((SPB, RSPLIT, 128), lambda i, q=q: (i, q, 0))

    in_specs = [split_spec(q) for q in range(SPLITS)] * 3
    operands = ([xr1] * SPLITS) + ([xr2] * SPLITS) + ([x] * SPLITS)
    acc_spec = pl.BlockSpec((SPB, 1, 1), lambda i: (i, 0, 0))
    acc1, acc2 = pl.pallas_call(
        _acc_kernel,
        grid=(STEPS,),
        in_specs=in_specs,
        out_specs=[acc_spec, acc_spec],
        out_shape=[
            jax.ShapeDtypeStruct((N, 1, 1), jnp.float32),
            jax.ShapeDtypeStruct((N, 1, 1), jnp.float32),
        ],
        compiler_params=pltpu.CompilerParams(
            dimension_semantics=("arbitrary",),
        ),
    )(*operands)

    a1c = acc1.reshape(N, 1)
    a2c = acc2.reshape(N, 1)
    a1r = acc1.reshape(1, N)
    a2r = acc2.reshape(1, N)
    l1, l2 = pl.pallas_call(
        _select_kernel,
        out_shape=[
            jax.ShapeDtypeStruct((1, 1), jnp.float32),
            jax.ShapeDtypeStruct((1, 1), jnp.float32),
        ],
    )(a1c, a2c, a1r, a2r)
    return (l1.reshape(()), l2.reshape(()))


# speedup vs baseline: 1.0090x; 1.0090x over previous
import jax
import jax.numpy as jnp
from jax.experimental import pallas as pl
from jax.experimental.pallas import tpu as pltpu

N = 128
D = 3 * 224 * 224
ROWS = D // 128
SPB = 2
STEPS = N // SPB
NSLOT = 8
REM = int(N * (1.0 - 0.1))


def _acc_kernel(x1_hbm, x2_hbm, xx_hbm, acc1_ref, acc2_ref, b1, b2, bx, sems):
    def start_copies(s, slot):
        src = pl.ds(s * SPB, SPB)
        pltpu.make_async_copy(x1_hbm.at[src], b1.at[slot], sems.at[0, slot]).start()
        pltpu.make_async_copy(x2_hbm.at[src], b2.at[slot], sems.at[1, slot]).start()
        pltpu.make_async_copy(xx_hbm.at[src], bx.at[slot], sems.at[2, slot]).start()

    for s in range(NSLOT):
        start_copies(s, s)

    def body(s, _):
        slot = jax.lax.rem(s, NSLOT)
        src = pl.ds(s * SPB, SPB)
        pltpu.make_async_copy(x1_hbm.at[src], b1.at[slot], sems.at[0, slot]).wait()
        pltpu.make_async_copy(x2_hbm.at[src], b2.at[slot], sems.at[1, slot]).wait()
        pltpu.make_async_copy(xx_hbm.at[src], bx.at[slot], sems.at[2, slot]).wait()
        x = bx[slot]
        d1 = b1[slot] - x
        d2 = b2[slot] - x
        acc1_ref[s] = jnp.sum(d1 * d1, axis=(1, 2)).reshape(SPB, 1)
        acc2_ref[s] = jnp.sum(d2 * d2, axis=(1, 2)).reshape(SPB, 1)

        @pl.when(s + NSLOT < STEPS)
        def _():
            start_copies(s + NSLOT, slot)

        return 0

    jax.lax.fori_loop(0, STEPS, body, 0)


def _select_kernel(a1c_ref, a2c_ref, a1r_ref, a2r_ref, l1_ref, l2_ref):
    a1c = a1c_ref[...]
    a2c = a2c_ref[...]
    a1r = a1r_ref[...]
    a2r = a2r_ref[...]
    jidx = jax.lax.broadcasted_iota(jnp.int32, (N, N), 1)
    iidx = jax.lax.broadcasted_iota(jnp.int32, (N, N), 0)
    tie = jidx < iidx
    cmp2 = (a2r < a2c) | ((a2r == a2c) & tie)
    cmp1 = (a1r < a1c) | ((a1r == a1c) & tie)
    rank2 = jnp.sum(cmp2.astype(jnp.int32), axis=1, keepdims=True)
    rank1 = jnp.sum(cmp1.astype(jnp.int32), axis=1, keepdims=True)
    sel2 = rank2 < REM
    sel1 = rank1 < REM
    scale = 1.0 / (REM * D)
    l1_ref[...] = jnp.sum(jnp.where(sel2, a1c, 0.0), axis=0, keepdims=True) * scale
    l2_ref[...] = jnp.sum(jnp.where(sel1, a2c, 0.0), axis=0, keepdims=True) * scale


def kernel(xr1, xr2, x):
    xr1 = xr1.reshape(N, ROWS, 128)
    xr2 = xr2.reshape(N, ROWS, 128)
    x = x.reshape(N, ROWS, 128)

    any_spec = pl.BlockSpec(memory_space=pl.ANY)
    acc1, acc2 = pl.pallas_call(
        _acc_kernel,
        in_specs=[any_spec, any_spec, any_spec],
        out_shape=[
            jax.ShapeDtypeStruct((STEPS, SPB, 1), jnp.float32),
            jax.ShapeDtypeStruct((STEPS, SPB, 1), jnp.float32),
        ],
        scratch_shapes=[
            pltpu.VMEM((NSLOT, SPB, ROWS, 128), jnp.float32),
            pltpu.VMEM((NSLOT, SPB, ROWS, 128), jnp.float32),
            pltpu.VMEM((NSLOT, SPB, ROWS, 128), jnp.float32),
            pltpu.SemaphoreType.DMA((3, NSLOT)),
        ],
    )(xr1, xr2, x)

    a1c = acc1.reshape(N, 1)
    a2c = acc2.reshape(N, 1)
    a1r = acc1.reshape(1, N)
    a2r = acc2.reshape(1, N)
    l1, l2 = pl.pallas_call(
        _select_kernel,
        out_shape=[
            jax.ShapeDtypeStruct((1, 1), jnp.float32),
            jax.ShapeDtypeStruct((1, 1), jnp.float32),
        ],
    )(a1c, a2c, a1r, a2r)
    return (l1.reshape(()), l2.reshape(()))
